# Initial kernel scaffold; baseline (speedup 1.0000x reference)
import jax
import jax.numpy as jnp
from jax.experimental import pallas as pl


def kernel(inputs):
    raise NotImplementedError("write your pallas kernel here")


# speedup vs baseline: 23.1594x; 23.1594x over previous
import functools

import jax
import jax.numpy as jnp
from jax import lax
from jax.experimental import pallas as pl
from jax.experimental.pallas import tpu as pltpu
from jax.experimental.pallas import tpu_sc as plsc

N_TOKENS = 1000
BATCH = 16384
SEQ_LEN = 200
OUT_COLS = N_TOKENS - 1

_INFO = plsc.get_sparse_core_info()
NUM_CORES = _INFO.num_cores
NUM_SUBCORES = _INFO.num_subcores
LANES = _INFO.num_lanes
NW = NUM_CORES * NUM_SUBCORES

ROWS_PER_WORKER = BATCH // NW
CHUNK_ROWS = 64
NUM_CHUNKS = ROWS_PER_WORKER // CHUNK_ROWS

IN_CHUNK = CHUNK_ROWS * SEQ_LEN
OUT_CHUNK = CHUNK_ROWS * OUT_COLS
ZERO_GROUPS = OUT_CHUNK // LANES
TOK_GROUPS = IN_CHUNK // LANES


def _bow_body(in_hbm, out_hbm, in_v, counts_v):
    wid = lax.axis_index("s") * NUM_CORES + lax.axis_index("c")
    iota = lax.iota(jnp.int32, LANES)
    zeros = jnp.zeros((LANES,), jnp.float32)
    ones = jnp.ones((LANES,), jnp.float32)

    def zero_step(j, _):
        counts_v[pl.ds(j * LANES, LANES)] = zeros
        return 0

    def tok_step(g, _):
        toks = in_v[pl.ds(g * LANES, LANES)]
        pos = g * LANES + iota
        row = pos // SEQ_LEN
        idx = row * OUT_COLS + toks - 1
        plsc.addupdate_scatter(counts_v, [idx], ones, mask=toks != 0)
        return 0

    for c in range(NUM_CHUNKS):
        in_base = wid * ROWS_PER_WORKER * SEQ_LEN + c * IN_CHUNK
        out_base = wid * ROWS_PER_WORKER * OUT_COLS + c * OUT_CHUNK
        pltpu.sync_copy(in_hbm.at[pl.ds(in_base, IN_CHUNK)], in_v)
        lax.fori_loop(0, ZERO_GROUPS, zero_step, 0)
        lax.fori_loop(0, TOK_GROUPS, tok_step, 0)
        pltpu.sync_copy(counts_v, out_hbm.at[pl.ds(out_base, OUT_CHUNK)])


_bow_kernel = functools.partial(
    pl.kernel,
    out_type=jax.ShapeDtypeStruct((BATCH * OUT_COLS,), jnp.float32),
    mesh=plsc.VectorSubcoreMesh(core_axis_name="c", subcore_axis_name="s"),
    scratch_types=[
        pltpu.VMEM((IN_CHUNK,), jnp.int32),
        pltpu.VMEM((OUT_CHUNK,), jnp.float32),
    ],
    compiler_params=pltpu.CompilerParams(needs_layout_passes=False),
)(_bow_body)


@jax.jit
def kernel(inputs):
    flat = inputs.reshape(-1)
    out = _bow_kernel(flat)
    return out.reshape(BATCH, OUT_COLS)


# speedup vs baseline: 39.7935x; 1.7182x over previous
import functools

import jax
import jax.numpy as jnp
from jax import lax
from jax.experimental import pallas as pl
from jax.experimental.pallas import tpu as pltpu
from jax.experimental.pallas import tpu_sc as plsc

N_TOKENS = 1000
BATCH = 16384
SEQ_LEN = 200
OUT_COLS = N_TOKENS - 1

_INFO = plsc.get_sparse_core_info()
NUM_CORES = _INFO.num_cores
NUM_SUBCORES = _INFO.num_subcores
LANES = _INFO.num_lanes
NW = NUM_CORES * NUM_SUBCORES

ROWS_PER_WORKER = BATCH // NW
CHUNK_ROWS = 64
NUM_CHUNKS = ROWS_PER_WORKER // CHUNK_ROWS

IN_CHUNK = CHUNK_ROWS * SEQ_LEN
OUT_CHUNK = CHUNK_ROWS * OUT_COLS
ZERO_GROUPS = OUT_CHUNK // LANES
TOK_GROUPS = IN_CHUNK // LANES


def _bow_body(in_hbm, out_hbm, in_v, counts_v):
    wid = lax.axis_index("s") * NUM_CORES + lax.axis_index("c")
    iota = lax.iota(jnp.int32, LANES)
    zeros = jnp.zeros((LANES,), jnp.float32)
    ones = jnp.ones((LANES,), jnp.float32)

    def zero_step(j):
        counts_v[pl.ds(j * LANES, LANES)] = zeros

    def tok_step(g):
        toks = in_v[pl.ds(g * LANES, LANES)]
        pos = g * LANES + iota
        row = pos // SEQ_LEN
        idx = row * OUT_COLS + toks - 1
        plsc.addupdate_scatter(counts_v, [idx], ones, mask=toks != 0)

    for c in range(NUM_CHUNKS):
        in_base = wid * ROWS_PER_WORKER * SEQ_LEN + c * IN_CHUNK
        out_base = wid * ROWS_PER_WORKER * OUT_COLS + c * OUT_CHUNK
        pltpu.sync_copy(in_hbm.at[pl.ds(in_base, IN_CHUNK)], in_v)
        plsc.parallel_loop(0, ZERO_GROUPS, unroll=12)(zero_step)
        plsc.parallel_loop(0, TOK_GROUPS, unroll=8)(tok_step)
        pltpu.sync_copy(counts_v, out_hbm.at[pl.ds(out_base, OUT_CHUNK)])


_bow_kernel = functools.partial(
    pl.kernel,
    out_type=jax.ShapeDtypeStruct((BATCH * OUT_COLS,), jnp.float32),
    mesh=plsc.VectorSubcoreMesh(core_axis_name="c", subcore_axis_name="s"),
    scratch_types=[
        pltpu.VMEM((IN_CHUNK,), jnp.int32),
        pltpu.VMEM((OUT_CHUNK,), jnp.float32),
    ],
    compiler_params=pltpu.CompilerParams(needs_layout_passes=False),
)(_bow_body)


@jax.jit
def kernel(inputs):
    flat = inputs.reshape(-1)
    out = _bow_kernel(flat)
    return out.reshape(BATCH, OUT_COLS)


# speedup vs baseline: 74.2487x; 1.8659x over previous
import functools

import jax
import jax.numpy as jnp
from jax import lax
from jax.experimental import pallas as pl
from jax.experimental.pallas import tpu as pltpu
from jax.experimental.pallas import tpu_sc as plsc

N_TOKENS = 1000
BATCH = 16384
SEQ_LEN = 200
OUT_COLS = N_TOKENS - 1

_INFO = plsc.get_sparse_core_info()
NUM_CORES = _INFO.num_cores
NUM_SUBCORES = _INFO.num_subcores
LANES = _INFO.num_lanes
NW = NUM_CORES * NUM_SUBCORES

STRIPE = 128
SPW = BATCH // (NW * STRIPE)
SEQ_TILES = SEQ_LEN // 8
HALF0 = 496
HALF1 = OUT_COLS - HALF0
TILE_GROUPS = 8 * STRIPE // LANES


def _bow_body(in_hbm, out_hbm, ina, inb, cnt0, cnt1,
              in_sem_a, in_sem_b, out_sem0, out_sem1):
    wid = lax.axis_index("s") * NUM_CORES + lax.axis_index("c")
    iota = lax.iota(jnp.int32, LANES)
    zeros = jnp.zeros((LANES,), jnp.float32)
    ones = jnp.ones((LANES,), jnp.float32)

    cnts = (cnt0, cnt1)
    out_sems = (out_sem0, out_sem1)
    halves = ((0, HALF0), (HALF0, HALF1))
    base = wid * SPW * STRIPE

    def make_zero(cnt):
        def zero_step(j):
            cnt[j >> 3, pl.ds((j & 7) * LANES, LANES)] = zeros
        return zero_step

    def make_scatter(inb_, cnt, r0, nrows):
        lo = r0 + 1
        hi = r0 + nrows

        def tok_step(g):
            k = (g & 7) * LANES
            toks = inb_[g >> 3, pl.ds(k, LANES)]
            mask = (toks >= lo) & (toks <= hi)
            plsc.addupdate_scatter(
                cnt, [toks - lo, k + iota], ones, mask=mask)
        return tok_step

    out_cp = {}
    for u in range(SPW * 2):
        s, h = u // 2, u % 2
        r0, nrows = halves[h]
        cnt = cnts[h]
        col = pl.ds(base + s * STRIPE, STRIPE)

        def start(t, buf, sem):
            return pltpu.async_copy(
                in_hbm.at[pl.ds(t * 8, 8), col], buf, sem)

        def wait(buf, sem):
            pltpu.make_async_copy(
                in_hbm.at[pl.ds(0, 8), col], buf, sem).wait()

        if u >= 2:
            out_cp[u - 2].wait()
        plsc.parallel_loop(0, nrows * 8, unroll=12)(make_zero(cnt))

        scat_a = make_scatter(ina, cnt, r0, nrows)
        scat_b = make_scatter(inb, cnt, r0, nrows)
        start(0, ina, in_sem_a)

        @pl.loop(0, SEQ_TILES // 2)
        def tile_pair(i):
            t = i * 2
            start(t + 1, inb, in_sem_b)
            wait(ina, in_sem_a)
            plsc.parallel_loop(0, TILE_GROUPS, unroll=8)(scat_a)
            start(t + 2, ina, in_sem_a)
            wait(inb, in_sem_b)
            plsc.parallel_loop(0, TILE_GROUPS, unroll=8)(scat_b)

        wait(ina, in_sem_a)
        plsc.parallel_loop(0, TILE_GROUPS, unroll=8)(scat_a)

        out_cp[u] = pltpu.async_copy(
            cnt, out_hbm.at[pl.ds(r0, nrows), col], out_sems[h])
    out_cp[SPW * 2 - 2].wait()
    out_cp[SPW * 2 - 1].wait()


_bow_kernel = functools.partial(
    pl.kernel,
    out_type=jax.ShapeDtypeStruct((OUT_COLS, BATCH), jnp.float32),
    mesh=plsc.VectorSubcoreMesh(core_axis_name="c", subcore_axis_name="s"),
    scratch_types=[
        pltpu.VMEM((8, STRIPE), jnp.int32),
        pltpu.VMEM((8, STRIPE), jnp.int32),
        pltpu.VMEM((HALF0, STRIPE), jnp.float32),
        pltpu.VMEM((HALF1, STRIPE), jnp.float32),
        pltpu.SemaphoreType.DMA,
        pltpu.SemaphoreType.DMA,
        pltpu.SemaphoreType.DMA,
        pltpu.SemaphoreType.DMA,
    ],
    compiler_params=pltpu.CompilerParams(
        needs_layout_passes=False,
        use_tc_tiling_on_sc=True,
    ),
)(_bow_body)


@jax.jit
def kernel(inputs):
    out_t = _bow_kernel(inputs.T)
    return out_t.T
